# Initial kernel scaffold; baseline (speedup 1.0000x reference)
import jax
import jax.numpy as jnp
from jax.experimental import pallas as pl


def kernel(inp, GN):
    raise NotImplementedError("write your pallas kernel here")


# speedup vs baseline: 8.1627x; 8.1627x over previous
import functools

import jax
import jax.numpy as jnp
from jax.experimental import pallas as pl

_BS = 4
_D0 = 128
_D1 = 8192
_K = 8


def _topk_onehot_body(inp_ref, gn_ref, out_ref, *, rows):
    x = inp_ref[...]
    g = gn_ref[0]
    p = x + g
    col = jax.lax.broadcasted_iota(jnp.int32, (rows, _D1), 1)

    work = p
    idxs = []
    for _ in range(_K):
        m = jnp.max(work, axis=-1, keepdims=True)
        am = jnp.min(jnp.where(work == m, col, _D1), axis=-1)
        idxs.append(am)
        work = jnp.where(col == am[:, None], -jnp.inf, work)

    cnt = jnp.zeros((rows, _D1), dtype=jnp.int32)
    mask = jnp.zeros((rows, _D1), dtype=jnp.bool_)
    for am in idxs:
        a = am[:, None]
        cnt = cnt + (a < col).astype(jnp.int32)
        mask = mask | (col == a)

    code = jnp.where(mask, cnt, -1)
    j_iota = jax.lax.broadcasted_iota(jnp.int32, (rows, _K, _D1), 1)
    out_ref[0] = (code[:, None, :] == j_iota).astype(jnp.float32)


@functools.partial(jax.jit, static_argnames=("rows",))
def _run(inp, GN, rows=8):
    grid = (_BS, _D0 // rows)
    return pl.pallas_call(
        functools.partial(_topk_onehot_body, rows=rows),
        grid=grid,
        in_specs=[
            pl.BlockSpec((rows, _D1), lambda b, i: (i, 0)),
            pl.BlockSpec((1, rows, _D1), lambda b, i: (b, i, 0)),
        ],
        out_specs=pl.BlockSpec((1, rows, _K, _D1), lambda b, i: (b, i, 0, 0)),
        out_shape=jax.ShapeDtypeStruct((_BS, _D0, _K, _D1), jnp.float32),
    )(inp, GN)


def kernel(inp, GN):
    return _run(inp, GN)


# speedup vs baseline: 16.5594x; 2.0287x over previous
import functools

import jax
import jax.numpy as jnp
from jax.experimental import pallas as pl

_BS = 4
_D0 = 128
_D1 = 8192
_K = 8

_SORT8 = [
    (0, 1), (2, 3), (4, 5), (6, 7),
    (0, 2), (1, 3), (4, 6), (5, 7),
    (1, 2), (5, 6), (0, 4), (3, 7),
    (1, 5), (2, 6),
    (1, 4), (3, 6),
    (2, 4), (3, 5),
    (3, 4),
]


def _topk_onehot_body(inp_ref, gn_ref, out_ref, *, rows):
    x = inp_ref[...]
    g = gn_ref[0]
    p = x + g
    col = jax.lax.broadcasted_iota(jnp.int32, (rows, _D1), 1)

    work = p
    idxs = []
    for _ in range(_K):
        m = jnp.max(work, axis=-1, keepdims=True)
        am = jnp.min(jnp.where(work == m, col, _D1), axis=-1)
        idxs.append(am)
        work = jnp.where(col == am[:, None], -jnp.inf, work)

    for a, b in _SORT8:
        lo = jnp.minimum(idxs[a], idxs[b])
        hi = jnp.maximum(idxs[a], idxs[b])
        idxs[a], idxs[b] = lo, hi

    for j in range(_K):
        out_ref[0, :, j, :] = (col == idxs[j][:, None]).astype(jnp.float32)


@functools.partial(jax.jit, static_argnames=("rows",))
def _run(inp, GN, rows=64):
    grid = (_BS, _D0 // rows)
    return pl.pallas_call(
        functools.partial(_topk_onehot_body, rows=rows),
        grid=grid,
        in_specs=[
            pl.BlockSpec((rows, _D1), lambda b, i: (i, 0)),
            pl.BlockSpec((1, rows, _D1), lambda b, i: (b, i, 0)),
        ],
        out_specs=pl.BlockSpec((1, rows, _K, _D1), lambda b, i: (b, i, 0, 0)),
        out_shape=jax.ShapeDtypeStruct((_BS, _D0, _K, _D1), jnp.float32),
    )(inp, GN)


def kernel(inp, GN):
    return _run(inp, GN)
